# Initial kernel scaffold; baseline (speedup 1.0000x reference)
import jax
import jax.numpy as jnp
from jax.experimental import pallas as pl


def kernel(x, pos, batch, sa1, sa2, sa3, fp3, fp2, fp1, head):
    raise NotImplementedError("write your pallas kernel here")


# speedup vs baseline: 10.9969x; 10.9969x over previous
import functools

import jax
import jax.numpy as jnp
import numpy as np
from jax import lax
from jax.experimental import pallas as pl
from jax.experimental.pallas import tpu as pltpu
from jax.experimental.pallas import tpu_sc as plsc

B = 16
P = 2048
N = B * P
K = 32
S1 = 409
SP1 = 512
S2 = 102
SP2 = 128
BIG = np.float32(1e10)
F32 = np.float32


def _fps_body(n_real, nsamp, px_ref, py_ref, pz_ref,
              idx_ref, sx_ref, sy_ref, sz_ref):
    Bb, Pp = px_ref.shape
    SP = idx_ref.shape[1]
    px = px_ref[...]
    py = py_ref[...]
    pz = pz_ref[...]
    iota_p = lax.broadcasted_iota(jnp.int32, (Bb, Pp), 1)
    iota_s = lax.broadcasted_iota(jnp.int32, (Bb, SP), 1)
    dists0 = jnp.where(iota_p < n_real, BIG, F32(-1.0))
    lx0 = px[:, 0]
    ly0 = py[:, 0]
    lz0 = pz[:, 0]
    idx0 = jnp.zeros((Bb, SP), jnp.int32)
    sx0 = jnp.where(iota_s == 0, lx0[:, None], F32(0.0))
    sy0 = jnp.where(iota_s == 0, ly0[:, None], F32(0.0))
    sz0 = jnp.where(iota_s == 0, lz0[:, None], F32(0.0))

    def body(i, carry):
        dists, lx, ly, lz, idx, sx, sy, sz = carry
        d = ((px - lx[:, None]) ** 2 + (py - ly[:, None]) ** 2
             + (pz - lz[:, None]) ** 2)
        dists = jnp.minimum(dists, d)
        m = jnp.max(dists, axis=-1, keepdims=True)
        nxt = jnp.min(jnp.where(dists == m, iota_p, Pp), axis=-1)
        oh = iota_p == nxt[:, None]
        nlx = jnp.max(jnp.where(oh, px, -BIG), axis=-1)
        nly = jnp.max(jnp.where(oh, py, -BIG), axis=-1)
        nlz = jnp.max(jnp.where(oh, pz, -BIG), axis=-1)
        sel = iota_s == i
        idx = jnp.where(sel, nxt[:, None], idx)
        sx = jnp.where(sel, nlx[:, None], sx)
        sy = jnp.where(sel, nly[:, None], sy)
        sz = jnp.where(sel, nlz[:, None], sz)
        return dists, nlx, nly, nlz, idx, sx, sy, sz

    carry = lax.fori_loop(
        1, nsamp, body, (dists0, lx0, ly0, lz0, idx0, sx0, sy0, sz0))
    _, _, _, _, idx, sx, sy, sz = carry
    idx_ref[...] = idx.astype(jnp.int32)
    sx_ref[...] = sx
    sy_ref[...] = sy
    sz_ref[...] = sz


def _fps(px, py, pz, n_real, nsamp, SP):
    Bb, Pp = px.shape
    out_shape = (
        jax.ShapeDtypeStruct((Bb, SP), jnp.int32),
        jax.ShapeDtypeStruct((Bb, SP), F32),
        jax.ShapeDtypeStruct((Bb, SP), F32),
        jax.ShapeDtypeStruct((Bb, SP), F32),
    )
    return pl.pallas_call(
        functools.partial(_fps_body, n_real, nsamp),
        out_shape=out_shape,
    )(px, py, pz)


def _ball_topk_body(r2, n_real, stride, cx_ref, cy_ref, cz_ref,
                    px_ref, py_ref, pz_ref, nidx_ref, vm_ref):
    b = pl.program_id(0)
    SP = cx_ref.shape[2]
    Pp = px_ref.shape[2]
    cx = cx_ref[0, 0, :]
    cy = cy_ref[0, 0, :]
    cz = cz_ref[0, 0, :]
    px = px_ref[0, 0, :]
    py = py_ref[0, 0, :]
    pz = pz_ref[0, 0, :]
    d = ((cx[:, None] - px[None, :]) ** 2 + (cy[:, None] - py[None, :]) ** 2
         + (cz[:, None] - pz[None, :]) ** 2)
    iota_p = lax.broadcasted_iota(jnp.int32, (SP, Pp), 1)
    dm = jnp.where((d <= r2) & (iota_p < n_real), d, BIG)
    iota_k = lax.broadcasted_iota(jnp.int32, (SP, K), 1)
    nacc = jnp.zeros((SP, K), jnp.int32)
    vacc = jnp.zeros((SP, K), F32)
    for k in range(K):
        rmin = jnp.min(dm, axis=-1, keepdims=True)
        sel = jnp.min(jnp.where(dm == rmin, iota_p, Pp), axis=-1)
        valid = (rmin[:, 0] < F32(5e9)).astype(F32)
        lk = iota_k == k
        nacc = jnp.where(lk, sel[:, None], nacc)
        vacc = jnp.where(lk, valid[:, None], vacc)
        dm = jnp.where(iota_p == sel[:, None], BIG, dm)
    nidx_ref[0] = (nacc + b * stride).astype(jnp.int32)
    vm_ref[0] = vacc


def _ball_topk(cx, cy, cz, px, py, pz, r2, n_real, stride):
    Bb, SP = cx.shape
    Pp = px.shape[1]
    cx3 = cx.reshape(Bb, 1, SP)
    cy3 = cy.reshape(Bb, 1, SP)
    cz3 = cz.reshape(Bb, 1, SP)
    px3 = px.reshape(Bb, 1, Pp)
    py3 = py.reshape(Bb, 1, Pp)
    pz3 = pz.reshape(Bb, 1, Pp)
    spec_c = pl.BlockSpec((1, 1, SP), lambda b: (b, 0, 0))
    spec_p = pl.BlockSpec((1, 1, Pp), lambda b: (b, 0, 0))
    out_shape = (
        jax.ShapeDtypeStruct((Bb, SP, K), jnp.int32),
        jax.ShapeDtypeStruct((Bb, SP, K), F32),
    )
    out_spec = pl.BlockSpec((1, SP, K), lambda b: (b, 0, 0))
    return pl.pallas_call(
        functools.partial(_ball_topk_body, r2, n_real, stride),
        grid=(Bb,),
        in_specs=[spec_c, spec_c, spec_c, spec_p, spec_p, spec_p],
        out_specs=(out_spec, out_spec),
        out_shape=out_shape,
    )(cx3, cy3, cz3, px3, py3, pz3)


def _gather_rows(table, idx):
    M = idx.shape[0]
    Dp = table.shape[1]
    info = plsc.get_sparse_core_info()
    NC, NS = info.num_cores, info.num_subcores
    NW = NC * NS
    assert M % (NW * 128) == 0
    CPW = M // (NW * 128)
    mesh = plsc.VectorSubcoreMesh(core_axis_name="c", subcore_axis_name="s")

    @functools.partial(
        pl.kernel,
        out_type=jax.ShapeDtypeStruct((M, Dp), F32),
        mesh=mesh,
        compiler_params=pltpu.CompilerParams(use_tc_tiling_on_sc=False),
        scratch_types=[
            pltpu.VMEM((CPW * 128,), jnp.int32),
            pltpu.VMEM((128, Dp), F32),
            pltpu.SemaphoreType.DMA,
        ],
    )
    def gk(idx_hbm, table_hbm, out_hbm, idx_v, rows_v, sem):
        wid = lax.axis_index("s") * NC + lax.axis_index("c")
        base = wid * (CPW * 128)
        pltpu.sync_copy(idx_hbm.at[pl.ds(base, CPW * 128)], idx_v)

        def body(c, _):
            pltpu.async_copy(
                table_hbm.at[idx_v.at[pl.ds(c * 128, 128)]], rows_v, sem
            ).wait()
            pltpu.sync_copy(rows_v, out_hbm.at[pl.ds(base + c * 128, 128)])
            return 0

        lax.fori_loop(0, CPW, body, 0)

    return gk(idx, table)


def _sa_mlp_body(SBLK, g_ref, c_ref, vm_ref, w1_ref, b1_ref, w2_ref, b2_ref,
                 w3_ref, b3_ref, out_ref):
    g = g_ref[0] - c_ref[0]
    h = jnp.maximum(jnp.dot(g, w1_ref[...],
                            preferred_element_type=F32) + b1_ref[0], 0.0)
    h = jnp.maximum(jnp.dot(h, w2_ref[...],
                            preferred_element_type=F32) + b2_ref[0], 0.0)
    h = jnp.maximum(jnp.dot(h, w3_ref[...],
                            preferred_element_type=F32) + b3_ref[0], 0.0)
    C = h.shape[-1]
    h3 = h.reshape(SBLK, K, C)
    vm = vm_ref[0]
    h3 = jnp.where(vm[..., None] > 0, h3, -BIG)
    out_ref[0] = jnp.max(h3, axis=1)


def _sa_mlp_max(g, cpad, vm, w1, b1, w2, b2, w3, b3, SP, SBLK):
    Dp = g.shape[-1]
    C = w3.shape[1]
    nblk = SP // SBLK
    grid = (B, nblk)
    spec_g = pl.BlockSpec((1, SBLK * K, Dp), lambda b, s: (b, s, 0))
    spec_vm = pl.BlockSpec((1, SBLK, K), lambda b, s: (b, s, 0))
    spec_w = lambda sh: pl.BlockSpec(sh, lambda b, s: (0, 0))
    out_spec = pl.BlockSpec((1, SBLK, C), lambda b, s: (b, s, 0))
    return pl.pallas_call(
        functools.partial(_sa_mlp_body, SBLK),
        grid=grid,
        in_specs=[spec_g, spec_g, spec_vm,
                  spec_w(w1.shape), spec_w((1, w1.shape[1])),
                  spec_w(w2.shape), spec_w((1, w2.shape[1])),
                  spec_w(w3.shape), spec_w((1, w3.shape[1]))],
        out_specs=out_spec,
        out_shape=jax.ShapeDtypeStruct((B, SP, C), F32),
    )(g, cpad, vm, w1, b1.reshape(1, -1), w2, b2.reshape(1, -1),
      w3, b3.reshape(1, -1))


def _sa3fp3_body(x2_ref, p2x_ref, p2y_ref, p2z_ref,
                 w1a_ref, w1b_ref, b1_ref, w2_ref, b2_ref, w3_ref, b3_ref,
                 fw1a_ref, fw1b_ref, fb1_ref, fw2_ref, fb2_ref, out_ref):
    x2 = x2_ref[0]
    pcat = jnp.concatenate(
        [p2x_ref[0, 0, :][:, None], p2y_ref[0, 0, :][:, None],
         p2z_ref[0, 0, :][:, None]], axis=-1)
    h = jnp.dot(x2, w1a_ref[...], preferred_element_type=F32)
    h = h + jnp.dot(pcat, w1b_ref[...], preferred_element_type=F32)
    h = jnp.maximum(h + b1_ref[0], 0.0)
    h = jnp.maximum(jnp.dot(h, w2_ref[...],
                            preferred_element_type=F32) + b2_ref[0], 0.0)
    h = jnp.maximum(jnp.dot(h, w3_ref[...],
                            preferred_element_type=F32) + b3_ref[0], 0.0)
    rows = lax.broadcasted_iota(jnp.int32, h.shape, 0)
    h = jnp.where(rows < S2, h, -BIG)
    g = jnp.max(h, axis=0, keepdims=True)
    t = jnp.dot(g, fw1a_ref[...], preferred_element_type=F32)
    t = t + jnp.dot(x2, fw1b_ref[...], preferred_element_type=F32)
    t = jnp.maximum(t + fb1_ref[0], 0.0)
    f3 = jnp.maximum(jnp.dot(t, fw2_ref[...],
                             preferred_element_type=F32) + fb2_ref[0], 0.0)
    out_ref[0] = f3


def _sa3fp3(x2, p2x, p2y, p2z, sa3, fp3):
    (w1, b1), (w2, b2), (w3, b3) = sa3
    (fw1, fb1), (fw2, fb2) = fp3
    w1a, w1b = w1[:256], w1[256:259]
    fw1a, fw1b = fw1[:1024], fw1[1024:1280]
    spec_x = pl.BlockSpec((1, SP2, 256), lambda b: (b, 0, 0))
    spec_p = pl.BlockSpec((1, 1, SP2), lambda b: (b, 0, 0))
    spec_w = lambda sh: pl.BlockSpec(sh, lambda b: (0, 0))
    p3 = lambda a: a.reshape(B, 1, SP2)
    return pl.pallas_call(
        _sa3fp3_body,
        grid=(B,),
        in_specs=[spec_x, spec_p, spec_p, spec_p,
                  spec_w(w1a.shape), spec_w(w1b.shape), spec_w((1, 256)),
                  spec_w(w2.shape), spec_w((1, 512)),
                  spec_w(w3.shape), spec_w((1, 1024)),
                  spec_w(fw1a.shape), spec_w(fw1b.shape), spec_w((1, 256)),
                  spec_w(fw2.shape), spec_w((1, 256))],
        out_specs=pl.BlockSpec((1, SP2, 256), lambda b: (b, 0, 0)),
        out_shape=jax.ShapeDtypeStruct((B, SP2, 256), F32),
    )(x2, p3(p2x), p3(p2y), p3(p2z),
      w1a, w1b, b1.reshape(1, -1), w2, b2.reshape(1, -1),
      w3, b3.reshape(1, -1), fw1a, fw1b, fb1.reshape(1, -1),
      fw2, fb2.reshape(1, -1))


def _knn3_weight_matrix(tx, ty, tz, sx, sy, sz, n_src):
    T = tx.shape[0]
    S = sx.shape[0]
    d = ((tx[:, None] - sx[None, :]) ** 2 + (ty[:, None] - sy[None, :]) ** 2
         + (tz[:, None] - sz[None, :]) ** 2)
    iota_c = lax.broadcasted_iota(jnp.int32, (T, S), 1)
    d = jnp.where(iota_c < n_src, d, BIG)
    sels = []
    ws = []
    for _ in range(3):
        rmin = jnp.min(d, axis=-1, keepdims=True)
        sel = jnp.min(jnp.where(d == rmin, iota_c, S), axis=-1)
        w = 1.0 / (jnp.maximum(rmin[:, 0], 0.0) + F32(1e-8))
        sels.append(sel)
        ws.append(w)
        d = jnp.where(iota_c == sel[:, None], BIG, d)
    wsum = ws[0] + ws[1] + ws[2]
    wmat = jnp.zeros((T, S), F32)
    for sel, w in zip(sels, ws):
        wn = w / wsum
        wmat = wmat + jnp.where(iota_c == sel[:, None], wn[:, None], 0.0)
    return wmat


def _interp_fp2_body(p1x_ref, p1y_ref, p1z_ref, p2x_ref, p2y_ref, p2z_ref,
                     f3_ref, x1_ref, w1a_ref, w1b_ref, b1_ref,
                     w2_ref, b2_ref, out_ref):
    wmat = _knn3_weight_matrix(
        p1x_ref[0, 0, :], p1y_ref[0, 0, :], p1z_ref[0, 0, :],
        p2x_ref[0, 0, :], p2y_ref[0, 0, :], p2z_ref[0, 0, :], S2)
    interp = jnp.dot(wmat, f3_ref[0], preferred_element_type=F32)
    h = jnp.dot(interp, w1a_ref[...], preferred_element_type=F32)
    h = h + jnp.dot(x1_ref[0], w1b_ref[...], preferred_element_type=F32)
    h = jnp.maximum(h + b1_ref[0], 0.0)
    f2 = jnp.maximum(jnp.dot(h, w2_ref[...],
                             preferred_element_type=F32) + b2_ref[0], 0.0)
    out_ref[0] = f2


def _interp_fp2(p1x, p1y, p1z, p2x, p2y, p2z, f3, x1, fp2):
    (w1, b1), (w2, b2) = fp2
    w1a, w1b = w1[:256], w1[256:384]
    spec_p1 = pl.BlockSpec((1, 1, SP1), lambda b: (b, 0, 0))
    spec_p2 = pl.BlockSpec((1, 1, SP2), lambda b: (b, 0, 0))
    spec_w = lambda sh: pl.BlockSpec(sh, lambda b: (0, 0))
    r1 = lambda a: a.reshape(B, 1, SP1)
    r2_ = lambda a: a.reshape(B, 1, SP2)
    return pl.pallas_call(
        _interp_fp2_body,
        grid=(B,),
        in_specs=[spec_p1, spec_p1, spec_p1, spec_p2, spec_p2, spec_p2,
                  pl.BlockSpec((1, SP2, 256), lambda b: (b, 0, 0)),
                  pl.BlockSpec((1, SP1, 128), lambda b: (b, 0, 0)),
                  spec_w(w1a.shape), spec_w(w1b.shape), spec_w((1, 256)),
                  spec_w(w2.shape), spec_w((1, 128))],
        out_specs=pl.BlockSpec((1, SP1, 128), lambda b: (b, 0, 0)),
        out_shape=jax.ShapeDtypeStruct((B, SP1, 128), F32),
    )(r1(p1x), r1(p1y), r1(p1z), r2_(p2x), r2_(p2y), r2_(p2z), f3, x1,
      w1a, w1b, b1.reshape(1, -1), w2, b2.reshape(1, -1))


def _interp_fp1_head_body(pbx_ref, pby_ref, pbz_ref, p1x_ref, p1y_ref,
                          p1z_ref, f2_ref, xb_ref,
                          w1a_ref, w1b_ref, b1_ref, w2_ref, b2_ref,
                          w3_ref, b3_ref, hw1_ref, hb1_ref, hw2_ref,
                          hb2_ref, hw3_ref, hb3_ref, out_ref):
    wmat = _knn3_weight_matrix(
        pbx_ref[0, 0, :], pby_ref[0, 0, :], pbz_ref[0, 0, :],
        p1x_ref[0, 0, :], p1y_ref[0, 0, :], p1z_ref[0, 0, :], S1)
    interp = jnp.dot(wmat, f2_ref[0], preferred_element_type=F32)
    h = jnp.dot(interp, w1a_ref[...], preferred_element_type=F32)
    h = h + jnp.dot(xb_ref[0], w1b_ref[...], preferred_element_type=F32)
    h = jnp.maximum(h + b1_ref[0], 0.0)
    h = jnp.maximum(jnp.dot(h, w2_ref[...],
                            preferred_element_type=F32) + b2_ref[0], 0.0)
    h = jnp.maximum(jnp.dot(h, w3_ref[...],
                            preferred_element_type=F32) + b3_ref[0], 0.0)
    h = jnp.maximum(jnp.dot(h, hw1_ref[...],
                            preferred_element_type=F32) + hb1_ref[0], 0.0)
    h = jnp.maximum(jnp.dot(h, hw2_ref[...],
                            preferred_element_type=F32) + hb2_ref[0], 0.0)
    h = jnp.dot(h, hw3_ref[...], preferred_element_type=F32) + hb3_ref[0]
    out_ref[0] = jnp.tanh(h)


def _interp_fp1_head(pbx, pby, pbz, p1x, p1y, p1z, f2, xb, fp1, head):
    (w1, b1), (w2, b2), (w3, b3) = fp1
    (hw1, hb1), (hw2, hb2), (hw3, hb3) = head
    w1a, w1b = w1[:128], w1[128:131]
    spec_pb = pl.BlockSpec((1, 1, P), lambda b: (b, 0, 0))
    spec_p1 = pl.BlockSpec((1, 1, SP1), lambda b: (b, 0, 0))
    spec_w = lambda sh: pl.BlockSpec(sh, lambda b: (0, 0))
    rb = lambda a: a.reshape(B, 1, P)
    r1 = lambda a: a.reshape(B, 1, SP1)
    return pl.pallas_call(
        _interp_fp1_head_body,
        grid=(B,),
        in_specs=[spec_pb, spec_pb, spec_pb, spec_p1, spec_p1, spec_p1,
                  pl.BlockSpec((1, SP1, 128), lambda b: (b, 0, 0)),
                  pl.BlockSpec((1, P, 3), lambda b: (b, 0, 0)),
                  spec_w(w1a.shape), spec_w(w1b.shape), spec_w((1, 128)),
                  spec_w(w2.shape), spec_w((1, 128)),
                  spec_w(w3.shape), spec_w((1, 128)),
                  spec_w(hw1.shape), spec_w((1, 512)),
                  spec_w(hw2.shape), spec_w((1, 256)),
                  spec_w(hw3.shape), spec_w((1, 32))],
        out_specs=pl.BlockSpec((1, P, 32), lambda b: (b, 0, 0)),
        out_shape=jax.ShapeDtypeStruct((B, P, 32), F32),
    )(rb(pbx), rb(pby), rb(pbz), r1(p1x), r1(p1y), r1(p1z), f2, xb,
      w1a, w1b, b1.reshape(1, -1), w2, b2.reshape(1, -1),
      w3, b3.reshape(1, -1), hw1, hb1.reshape(1, -1),
      hw2, hb2.reshape(1, -1), hw3, hb3.reshape(1, -1))


def _pad_rows(w, rows):
    out = jnp.zeros((rows, w.shape[1]), F32)
    return out.at[: w.shape[0]].set(w)


def kernel(x, pos, batch, sa1, sa2, sa3, fp3, fp2, fp1, head):
    xb = x.reshape(B, P, 3)
    pb = pos.reshape(B, P, 3)
    pbx, pby, pbz = pb[..., 0], pb[..., 1], pb[..., 2]

    idx1, s1x, s1y, s1z = _fps(pbx, pby, pbz, P, S1, SP1)
    nidx1, vm1 = _ball_topk(s1x, s1y, s1z, pbx, pby, pbz,
                            r2=np.float32(0.2 * 0.2), n_real=P, stride=P)
    table1 = jnp.concatenate(
        [x, pos, jnp.zeros((N, 10), F32)], axis=1)
    g1 = _gather_rows(table1, nidx1.reshape(-1))
    cpad1 = jnp.concatenate(
        [jnp.zeros((B, SP1, 3), F32), s1x[..., None], s1y[..., None],
         s1z[..., None], jnp.zeros((B, SP1, 10), F32)], axis=-1)
    cpad1 = jnp.repeat(cpad1, K, axis=1)
    (w11, b11), (w12, b12), (w13, b13) = sa1
    x1 = _sa_mlp_max(g1.reshape(B, SP1 * K, 16), cpad1, vm1,
                     _pad_rows(w11, 16), b11, w12, b12, w13, b13,
                     SP=SP1, SBLK=128)

    idx2, s2x, s2y, s2z = _fps(s1x, s1y, s1z, S1, S2, SP2)
    nidx2, vm2 = _ball_topk(s2x, s2y, s2z, s1x, s1y, s1z,
                            r2=np.float32(0.4 * 0.4), n_real=S1, stride=SP1)
    table2 = jnp.concatenate(
        [x1.reshape(B * SP1, 128), s1x.reshape(B * SP1, 1),
         s1y.reshape(B * SP1, 1), s1z.reshape(B * SP1, 1),
         jnp.zeros((B * SP1, 13), F32)], axis=1)
    g2 = _gather_rows(table2, nidx2.reshape(-1))
    cpad2 = jnp.concatenate(
        [jnp.zeros((B, SP2, 128), F32), s2x[..., None], s2y[..., None],
         s2z[..., None], jnp.zeros((B, SP2, 13), F32)], axis=-1)
    cpad2 = jnp.repeat(cpad2, K, axis=1)
    (w21, b21), (w22, b22), (w23, b23) = sa2
    x2 = _sa_mlp_max(g2.reshape(B, SP2 * K, 144), cpad2, vm2,
                     _pad_rows(w21, 144), b21, w22, b22, w23, b23,
                     SP=SP2, SBLK=128)

    f3 = _sa3fp3(x2, s2x, s2y, s2z, sa3, fp3)
    f2 = _interp_fp2(s1x, s1y, s1z, s2x, s2y, s2z, f3, x1, fp2)
    outb = _interp_fp1_head(pbx, pby, pbz, s1x, s1y, s1z, f2, xb, fp1, head)

    out = outb.reshape(N, 32)
    gidx = (idx1[:, :S1]
            + jnp.arange(B, dtype=jnp.int32)[:, None] * P).reshape(-1)
    return out, gidx


# speedup vs baseline: 13.8751x; 1.2617x over previous
import functools

import jax
import jax.numpy as jnp
import numpy as np
from jax import lax
from jax.experimental import pallas as pl
from jax.experimental.pallas import tpu as pltpu
from jax.experimental.pallas import tpu_sc as plsc

B = 16
P = 2048
N = B * P
K = 32
S1 = 409
SP1 = 512
S2 = 102
SP2 = 128
BIG = np.float32(1e10)
F32 = np.float32


def _fps_body(n_real, nsamp, px_ref, py_ref, pz_ref,
              idx_ref, sx_ref, sy_ref, sz_ref):
    Bb, Pp = px_ref.shape
    SP = idx_ref.shape[1]
    px = px_ref[...]
    py = py_ref[...]
    pz = pz_ref[...]
    iota_p = lax.broadcasted_iota(jnp.int32, (Bb, Pp), 1)
    iota_s = lax.broadcasted_iota(jnp.int32, (Bb, SP), 1)
    dists0 = jnp.where(iota_p < n_real, BIG, F32(-1.0))
    lx0 = px[:, 0]
    ly0 = py[:, 0]
    lz0 = pz[:, 0]
    idx0 = jnp.zeros((Bb, SP), jnp.int32)
    sx0 = jnp.where(iota_s == 0, lx0[:, None], F32(0.0))
    sy0 = jnp.where(iota_s == 0, ly0[:, None], F32(0.0))
    sz0 = jnp.where(iota_s == 0, lz0[:, None], F32(0.0))

    def body(i, carry):
        dists, lx, ly, lz, idx, sx, sy, sz = carry
        d = ((px - lx[:, None]) ** 2 + (py - ly[:, None]) ** 2
             + (pz - lz[:, None]) ** 2)
        dists = jnp.minimum(dists, d)
        nxt = jnp.argmax(dists, axis=-1).astype(jnp.int32)
        oh = iota_p == nxt[:, None]
        nlx = jnp.max(jnp.where(oh, px, -BIG), axis=-1)
        nly = jnp.max(jnp.where(oh, py, -BIG), axis=-1)
        nlz = jnp.max(jnp.where(oh, pz, -BIG), axis=-1)
        sel = iota_s == i
        idx = jnp.where(sel, nxt[:, None], idx)
        sx = jnp.where(sel, nlx[:, None], sx)
        sy = jnp.where(sel, nly[:, None], sy)
        sz = jnp.where(sel, nlz[:, None], sz)
        return dists, nlx, nly, nlz, idx, sx, sy, sz

    carry = lax.fori_loop(
        1, nsamp, body, (dists0, lx0, ly0, lz0, idx0, sx0, sy0, sz0))
    _, _, _, _, idx, sx, sy, sz = carry
    idx_ref[...] = idx.astype(jnp.int32)
    sx_ref[...] = sx
    sy_ref[...] = sy
    sz_ref[...] = sz


def _fps(px, py, pz, n_real, nsamp, SP):
    Bb, Pp = px.shape
    out_shape = (
        jax.ShapeDtypeStruct((Bb, SP), jnp.int32),
        jax.ShapeDtypeStruct((Bb, SP), F32),
        jax.ShapeDtypeStruct((Bb, SP), F32),
        jax.ShapeDtypeStruct((Bb, SP), F32),
    )
    return pl.pallas_call(
        functools.partial(_fps_body, n_real, nsamp),
        out_shape=out_shape,
    )(px, py, pz)


def _ball_topk_body(r2, n_real, stride, cx_ref, cy_ref, cz_ref,
                    px_ref, py_ref, pz_ref, nidx_ref, vc_ref):
    b = pl.program_id(0)
    SP = cx_ref.shape[2]
    Pp = px_ref.shape[2]
    cx = cx_ref[0, 0, :]
    cy = cy_ref[0, 0, :]
    cz = cz_ref[0, 0, :]
    px = px_ref[0, 0, :]
    py = py_ref[0, 0, :]
    pz = pz_ref[0, 0, :]
    d = ((cx[:, None] - px[None, :]) ** 2 + (cy[:, None] - py[None, :]) ** 2
         + (cz[:, None] - pz[None, :]) ** 2)
    iota_p = lax.broadcasted_iota(jnp.int32, (SP, Pp), 1)
    dm = jnp.where((d <= r2) & (iota_p < n_real), d, BIG)
    cnt = jnp.sum((dm < F32(5e9)).astype(jnp.int32), axis=-1)
    iota_k = lax.broadcasted_iota(jnp.int32, (SP, K), 1)
    nacc = jnp.zeros((SP, K), jnp.int32)
    for k in range(K):
        sel = jnp.argmin(dm, axis=-1).astype(jnp.int32)
        nacc = jnp.where(iota_k == k, sel[:, None], nacc)
        dm = jnp.where(iota_p == sel[:, None], BIG, dm)
    nidx_ref[0] = (nacc + b * stride).astype(jnp.int32)
    vc_ref[0, 0] = jnp.minimum(cnt, K)


def _ball_topk(cx, cy, cz, px, py, pz, r2, n_real, stride):
    Bb, SP = cx.shape
    Pp = px.shape[1]
    cx3 = cx.reshape(Bb, 1, SP)
    cy3 = cy.reshape(Bb, 1, SP)
    cz3 = cz.reshape(Bb, 1, SP)
    px3 = px.reshape(Bb, 1, Pp)
    py3 = py.reshape(Bb, 1, Pp)
    pz3 = pz.reshape(Bb, 1, Pp)
    spec_c = pl.BlockSpec((1, 1, SP), lambda b: (b, 0, 0))
    spec_p = pl.BlockSpec((1, 1, Pp), lambda b: (b, 0, 0))
    out_shape = (
        jax.ShapeDtypeStruct((Bb, SP, K), jnp.int32),
        jax.ShapeDtypeStruct((Bb, 1, SP), jnp.int32),
    )
    return pl.pallas_call(
        functools.partial(_ball_topk_body, r2, n_real, stride),
        grid=(Bb,),
        in_specs=[spec_c, spec_c, spec_c, spec_p, spec_p, spec_p],
        out_specs=(pl.BlockSpec((1, SP, K), lambda b: (b, 0, 0)),
                   pl.BlockSpec((1, 1, SP), lambda b: (b, 0, 0))),
        out_shape=out_shape,
    )(cx3, cy3, cz3, px3, py3, pz3)


def _gather_rows(table, idx):
    M = idx.shape[0]
    Dp = table.shape[1]
    info = plsc.get_sparse_core_info()
    NC, NS = info.num_cores, info.num_subcores
    NW = NC * NS
    assert M % (NW * 128) == 0
    CPW = M // (NW * 128)
    mesh = plsc.VectorSubcoreMesh(core_axis_name="c", subcore_axis_name="s")

    G = 1
    for cand in (2, 4, 8):
        if CPW % cand == 0 and cand * 128 * Dp * 4 <= 380 * 1024:
            G = cand
    NG = CPW // G

    @functools.partial(
        pl.kernel,
        out_type=jax.ShapeDtypeStruct((M, Dp), F32),
        mesh=mesh,
        compiler_params=pltpu.CompilerParams(use_tc_tiling_on_sc=False),
        scratch_types=[
            pltpu.VMEM((CPW * 128,), jnp.int32),
            pltpu.VMEM((G * 128, Dp), F32),
            pltpu.SemaphoreType.DMA,
        ],
    )
    def gk(idx_hbm, table_hbm, out_hbm, idx_v, rows_v, sem):
        wid = lax.axis_index("s") * NC + lax.axis_index("c")
        base = wid * (CPW * 128)
        pltpu.sync_copy(idx_hbm.at[pl.ds(base, CPW * 128)], idx_v)

        def body(g, _):
            descs = []
            for b in range(G):
                descs.append(pltpu.async_copy(
                    table_hbm.at[idx_v.at[pl.ds((g * G + b) * 128, 128)]],
                    rows_v.at[pl.ds(b * 128, 128)], sem))
            for dsc in descs:
                dsc.wait()
            pltpu.sync_copy(rows_v,
                            out_hbm.at[pl.ds(base + g * (G * 128), G * 128)])
            return 0

        lax.fori_loop(0, NG, body, 0)

    return gk(idx, table)


def _sa_mlp_body(SBLK, g_ref, c_ref, vc_ref, w1_ref, b1_ref, w2_ref, b2_ref,
                 w3_ref, b3_ref, out_ref):
    Dp = g_ref.shape[-1]
    cp = c_ref[0]
    g = (g_ref[0].reshape(SBLK, K, Dp) - cp[:, None, :]).reshape(SBLK * K, Dp)
    h = jnp.maximum(jnp.dot(g, w1_ref[...],
                            preferred_element_type=F32) + b1_ref[0], 0.0)
    h = jnp.maximum(jnp.dot(h, w2_ref[...],
                            preferred_element_type=F32) + b2_ref[0], 0.0)
    h = jnp.maximum(jnp.dot(h, w3_ref[...],
                            preferred_element_type=F32) + b3_ref[0], 0.0)
    C = h.shape[-1]
    h3 = h.reshape(SBLK, K, C)
    iota_k = lax.broadcasted_iota(jnp.int32, (SBLK, K), 1)
    vmf = (iota_k < vc_ref[0, 0][:, None]).astype(F32)
    h3 = jnp.where(vmf[:, :, None] > 0, h3, -BIG)
    out_ref[0] = jnp.max(h3, axis=1)


def _sa_mlp_max(g, cpad, vc, w1, b1, w2, b2, w3, b3, SP, SBLK):
    Dp = g.shape[-1]
    C = w3.shape[1]
    nblk = SP // SBLK
    grid = (B, nblk)
    spec_g = pl.BlockSpec((1, SBLK * K, Dp), lambda b, s: (b, s, 0))
    spec_c = pl.BlockSpec((1, SBLK, Dp), lambda b, s: (b, s, 0))
    spec_vc = pl.BlockSpec((1, 1, SBLK), lambda b, s: (b, 0, s))
    spec_w = lambda sh: pl.BlockSpec(sh, lambda b, s: (0, 0))
    out_spec = pl.BlockSpec((1, SBLK, C), lambda b, s: (b, s, 0))
    return pl.pallas_call(
        functools.partial(_sa_mlp_body, SBLK),
        grid=grid,
        in_specs=[spec_g, spec_c, spec_vc,
                  spec_w(w1.shape), spec_w((1, w1.shape[1])),
                  spec_w(w2.shape), spec_w((1, w2.shape[1])),
                  spec_w(w3.shape), spec_w((1, w3.shape[1]))],
        out_specs=out_spec,
        out_shape=jax.ShapeDtypeStruct((B, SP, C), F32),
    )(g, cpad, vc, w1, b1.reshape(1, -1), w2, b2.reshape(1, -1),
      w3, b3.reshape(1, -1))


def _sa3fp3_body(x2_ref, p2x_ref, p2y_ref, p2z_ref,
                 w1a_ref, w1b_ref, b1_ref, w2_ref, b2_ref, w3_ref, b3_ref,
                 fw1a_ref, fw1b_ref, fb1_ref, fw2_ref, fb2_ref, out_ref):
    x2 = x2_ref[0]
    pcat = jnp.concatenate(
        [p2x_ref[0, 0, :][:, None], p2y_ref[0, 0, :][:, None],
         p2z_ref[0, 0, :][:, None]], axis=-1)
    h = jnp.dot(x2, w1a_ref[...], preferred_element_type=F32)
    h = h + jnp.dot(pcat, w1b_ref[...], preferred_element_type=F32)
    h = jnp.maximum(h + b1_ref[0], 0.0)
    h = jnp.maximum(jnp.dot(h, w2_ref[...],
                            preferred_element_type=F32) + b2_ref[0], 0.0)
    h = jnp.maximum(jnp.dot(h, w3_ref[...],
                            preferred_element_type=F32) + b3_ref[0], 0.0)
    rows = lax.broadcasted_iota(jnp.int32, h.shape, 0)
    h = jnp.where(rows < S2, h, -BIG)
    g = jnp.max(h, axis=0, keepdims=True)
    t = jnp.dot(g, fw1a_ref[...], preferred_element_type=F32)
    t = t + jnp.dot(x2, fw1b_ref[...], preferred_element_type=F32)
    t = jnp.maximum(t + fb1_ref[0], 0.0)
    f3 = jnp.maximum(jnp.dot(t, fw2_ref[...],
                             preferred_element_type=F32) + fb2_ref[0], 0.0)
    out_ref[0] = f3


def _sa3fp3(x2, p2x, p2y, p2z, sa3, fp3):
    (w1, b1), (w2, b2), (w3, b3) = sa3
    (fw1, fb1), (fw2, fb2) = fp3
    w1a, w1b = w1[:256], w1[256:259]
    fw1a, fw1b = fw1[:1024], fw1[1024:1280]
    spec_x = pl.BlockSpec((1, SP2, 256), lambda b: (b, 0, 0))
    spec_p = pl.BlockSpec((1, 1, SP2), lambda b: (b, 0, 0))
    spec_w = lambda sh: pl.BlockSpec(sh, lambda b: (0, 0))
    p3 = lambda a: a.reshape(B, 1, SP2)
    return pl.pallas_call(
        _sa3fp3_body,
        grid=(B,),
        in_specs=[spec_x, spec_p, spec_p, spec_p,
                  spec_w(w1a.shape), spec_w(w1b.shape), spec_w((1, 256)),
                  spec_w(w2.shape), spec_w((1, 512)),
                  spec_w(w3.shape), spec_w((1, 1024)),
                  spec_w(fw1a.shape), spec_w(fw1b.shape), spec_w((1, 256)),
                  spec_w(fw2.shape), spec_w((1, 256))],
        out_specs=pl.BlockSpec((1, SP2, 256), lambda b: (b, 0, 0)),
        out_shape=jax.ShapeDtypeStruct((B, SP2, 256), F32),
    )(x2, p3(p2x), p3(p2y), p3(p2z),
      w1a, w1b, b1.reshape(1, -1), w2, b2.reshape(1, -1),
      w3, b3.reshape(1, -1), fw1a, fw1b, fb1.reshape(1, -1),
      fw2, fb2.reshape(1, -1))


def _knn3_weight_matrix(tx, ty, tz, sx, sy, sz, n_src):
    T = tx.shape[0]
    S = sx.shape[0]
    d = ((tx[:, None] - sx[None, :]) ** 2 + (ty[:, None] - sy[None, :]) ** 2
         + (tz[:, None] - sz[None, :]) ** 2)
    iota_c = lax.broadcasted_iota(jnp.int32, (T, S), 1)
    d = jnp.where(iota_c < n_src, d, BIG)
    sels = []
    ws = []
    for _ in range(3):
        rmin = jnp.min(d, axis=-1, keepdims=True)
        sel = jnp.min(jnp.where(d == rmin, iota_c, S), axis=-1)
        w = 1.0 / (jnp.maximum(rmin[:, 0], 0.0) + F32(1e-8))
        sels.append(sel)
        ws.append(w)
        d = jnp.where(iota_c == sel[:, None], BIG, d)
    wsum = ws[0] + ws[1] + ws[2]
    wmat = jnp.zeros((T, S), F32)
    for sel, w in zip(sels, ws):
        wn = w / wsum
        wmat = wmat + jnp.where(iota_c == sel[:, None], wn[:, None], 0.0)
    return wmat


def _interp_fp2_body(p1x_ref, p1y_ref, p1z_ref, p2x_ref, p2y_ref, p2z_ref,
                     f3_ref, x1_ref, w1a_ref, w1b_ref, b1_ref,
                     w2_ref, b2_ref, out_ref):
    wmat = _knn3_weight_matrix(
        p1x_ref[0, 0, :], p1y_ref[0, 0, :], p1z_ref[0, 0, :],
        p2x_ref[0, 0, :], p2y_ref[0, 0, :], p2z_ref[0, 0, :], S2)
    interp = jnp.dot(wmat, f3_ref[0], preferred_element_type=F32)
    h = jnp.dot(interp, w1a_ref[...], preferred_element_type=F32)
    h = h + jnp.dot(x1_ref[0], w1b_ref[...], preferred_element_type=F32)
    h = jnp.maximum(h + b1_ref[0], 0.0)
    f2 = jnp.maximum(jnp.dot(h, w2_ref[...],
                             preferred_element_type=F32) + b2_ref[0], 0.0)
    out_ref[0] = f2


def _interp_fp2(p1x, p1y, p1z, p2x, p2y, p2z, f3, x1, fp2):
    (w1, b1), (w2, b2) = fp2
    w1a, w1b = w1[:256], w1[256:384]
    spec_p1 = pl.BlockSpec((1, 1, SP1), lambda b: (b, 0, 0))
    spec_p2 = pl.BlockSpec((1, 1, SP2), lambda b: (b, 0, 0))
    spec_w = lambda sh: pl.BlockSpec(sh, lambda b: (0, 0))
    r1 = lambda a: a.reshape(B, 1, SP1)
    r2_ = lambda a: a.reshape(B, 1, SP2)
    return pl.pallas_call(
        _interp_fp2_body,
        grid=(B,),
        in_specs=[spec_p1, spec_p1, spec_p1, spec_p2, spec_p2, spec_p2,
                  pl.BlockSpec((1, SP2, 256), lambda b: (b, 0, 0)),
                  pl.BlockSpec((1, SP1, 128), lambda b: (b, 0, 0)),
                  spec_w(w1a.shape), spec_w(w1b.shape), spec_w((1, 256)),
                  spec_w(w2.shape), spec_w((1, 128))],
        out_specs=pl.BlockSpec((1, SP1, 128), lambda b: (b, 0, 0)),
        out_shape=jax.ShapeDtypeStruct((B, SP1, 128), F32),
    )(r1(p1x), r1(p1y), r1(p1z), r2_(p2x), r2_(p2y), r2_(p2z), f3, x1,
      w1a, w1b, b1.reshape(1, -1), w2, b2.reshape(1, -1))


def _interp_fp1_head_body(pbx_ref, pby_ref, pbz_ref, p1x_ref, p1y_ref,
                          p1z_ref, f2_ref, xb_ref,
                          w1a_ref, w1b_ref, b1_ref, w2_ref, b2_ref,
                          w3_ref, b3_ref, hw1_ref, hb1_ref, hw2_ref,
                          hb2_ref, hw3_ref, hb3_ref, out_ref):
    wmat = _knn3_weight_matrix(
        pbx_ref[0, 0, :], pby_ref[0, 0, :], pbz_ref[0, 0, :],
        p1x_ref[0, 0, :], p1y_ref[0, 0, :], p1z_ref[0, 0, :], S1)
    interp = jnp.dot(wmat, f2_ref[0], preferred_element_type=F32)
    h = jnp.dot(interp, w1a_ref[...], preferred_element_type=F32)
    h = h + jnp.dot(xb_ref[0], w1b_ref[...], preferred_element_type=F32)
    h = jnp.maximum(h + b1_ref[0], 0.0)
    h = jnp.maximum(jnp.dot(h, w2_ref[...],
                            preferred_element_type=F32) + b2_ref[0], 0.0)
    h = jnp.maximum(jnp.dot(h, w3_ref[...],
                            preferred_element_type=F32) + b3_ref[0], 0.0)
    h = jnp.maximum(jnp.dot(h, hw1_ref[...],
                            preferred_element_type=F32) + hb1_ref[0], 0.0)
    h = jnp.maximum(jnp.dot(h, hw2_ref[...],
                            preferred_element_type=F32) + hb2_ref[0], 0.0)
    h = jnp.dot(h, hw3_ref[...], preferred_element_type=F32) + hb3_ref[0]
    out_ref[0] = jnp.tanh(h)


def _interp_fp1_head(pbx, pby, pbz, p1x, p1y, p1z, f2, xb, fp1, head):
    (w1, b1), (w2, b2), (w3, b3) = fp1
    (hw1, hb1), (hw2, hb2), (hw3, hb3) = head
    w1a, w1b = w1[:128], w1[128:131]
    spec_pb = pl.BlockSpec((1, 1, P), lambda b: (b, 0, 0))
    spec_p1 = pl.BlockSpec((1, 1, SP1), lambda b: (b, 0, 0))
    spec_w = lambda sh: pl.BlockSpec(sh, lambda b: (0, 0))
    rb = lambda a: a.reshape(B, 1, P)
    r1 = lambda a: a.reshape(B, 1, SP1)
    return pl.pallas_call(
        _interp_fp1_head_body,
        grid=(B,),
        in_specs=[spec_pb, spec_pb, spec_pb, spec_p1, spec_p1, spec_p1,
                  pl.BlockSpec((1, SP1, 128), lambda b: (b, 0, 0)),
                  pl.BlockSpec((1, P, 3), lambda b: (b, 0, 0)),
                  spec_w(w1a.shape), spec_w(w1b.shape), spec_w((1, 128)),
                  spec_w(w2.shape), spec_w((1, 128)),
                  spec_w(w3.shape), spec_w((1, 128)),
                  spec_w(hw1.shape), spec_w((1, 512)),
                  spec_w(hw2.shape), spec_w((1, 256)),
                  spec_w(hw3.shape), spec_w((1, 32))],
        out_specs=pl.BlockSpec((1, P, 32), lambda b: (b, 0, 0)),
        out_shape=jax.ShapeDtypeStruct((B, P, 32), F32),
    )(rb(pbx), rb(pby), rb(pbz), r1(p1x), r1(p1y), r1(p1z), f2, xb,
      w1a, w1b, b1.reshape(1, -1), w2, b2.reshape(1, -1),
      w3, b3.reshape(1, -1), hw1, hb1.reshape(1, -1),
      hw2, hb2.reshape(1, -1), hw3, hb3.reshape(1, -1))


def _pad_rows(w, rows):
    out = jnp.zeros((rows, w.shape[1]), F32)
    return out.at[: w.shape[0]].set(w)


def kernel(x, pos, batch, sa1, sa2, sa3, fp3, fp2, fp1, head):
    xb = x.reshape(B, P, 3)
    pb = pos.reshape(B, P, 3)
    pbx, pby, pbz = pb[..., 0], pb[..., 1], pb[..., 2]

    idx1, s1x, s1y, s1z = _fps(pbx, pby, pbz, P, S1, SP1)
    nidx1, vc1 = _ball_topk(s1x, s1y, s1z, pbx, pby, pbz,
                            r2=np.float32(0.2 * 0.2), n_real=P, stride=P)
    table1 = jnp.concatenate(
        [x, pos, jnp.zeros((N, 10), F32)], axis=1)
    g1 = _gather_rows(table1, nidx1.reshape(-1))
    cpad1 = jnp.concatenate(
        [jnp.zeros((B, SP1, 3), F32), s1x[..., None], s1y[..., None],
         s1z[..., None], jnp.zeros((B, SP1, 10), F32)], axis=-1)
    (w11, b11), (w12, b12), (w13, b13) = sa1
    x1 = _sa_mlp_max(g1.reshape(B, SP1 * K, 16), cpad1, vc1,
                     _pad_rows(w11, 16), b11, w12, b12, w13, b13,
                     SP=SP1, SBLK=128)

    idx2, s2x, s2y, s2z = _fps(s1x, s1y, s1z, S1, S2, SP2)
    nidx2, vc2 = _ball_topk(s2x, s2y, s2z, s1x, s1y, s1z,
                            r2=np.float32(0.4 * 0.4), n_real=S1, stride=SP1)
    table2 = jnp.concatenate(
        [x1.reshape(B * SP1, 128), s1x.reshape(B * SP1, 1),
         s1y.reshape(B * SP1, 1), s1z.reshape(B * SP1, 1),
         jnp.zeros((B * SP1, 13), F32)], axis=1)
    g2 = _gather_rows(table2, nidx2.reshape(-1))
    cpad2 = jnp.concatenate(
        [jnp.zeros((B, SP2, 128), F32), s2x[..., None], s2y[..., None],
         s2z[..., None], jnp.zeros((B, SP2, 13), F32)], axis=-1)
    (w21, b21), (w22, b22), (w23, b23) = sa2
    x2 = _sa_mlp_max(g2.reshape(B, SP2 * K, 144), cpad2, vc2,
                     _pad_rows(w21, 144), b21, w22, b22, w23, b23,
                     SP=SP2, SBLK=128)

    f3 = _sa3fp3(x2, s2x, s2y, s2z, sa3, fp3)
    f2 = _interp_fp2(s1x, s1y, s1z, s2x, s2y, s2z, f3, x1, fp2)
    outb = _interp_fp1_head(pbx, pby, pbz, s1x, s1y, s1z, f2, xb, fp1, head)

    out = outb.reshape(N, 32)
    gidx = (idx1[:, :S1]
            + jnp.arange(B, dtype=jnp.int32)[:, None] * P).reshape(-1)
    return out, gidx


# speedup vs baseline: 14.0046x; 1.0093x over previous
import functools

import jax
import jax.numpy as jnp
import numpy as np
from jax import lax
from jax.experimental import pallas as pl
from jax.experimental.pallas import tpu as pltpu
from jax.experimental.pallas import tpu_sc as plsc

B = 16
P = 2048
N = B * P
K = 32
S1 = 409
SP1 = 512
S2 = 102
SP2 = 128
BIG = np.float32(1e10)
F32 = np.float32


def _fps_body(n_real, nsamp, px_ref, py_ref, pz_ref,
              idx_ref, sx_ref, sy_ref, sz_ref):
    Bb, Pp = px_ref.shape
    SP = idx_ref.shape[1]
    px = px_ref[...]
    py = py_ref[...]
    pz = pz_ref[...]
    iota_p = lax.broadcasted_iota(jnp.int32, (Bb, Pp), 1)
    iota_s = lax.broadcasted_iota(jnp.int32, (Bb, SP), 1)
    dists0 = jnp.where(iota_p < n_real, BIG, F32(-1.0))
    lx0 = px[:, 0]
    ly0 = py[:, 0]
    lz0 = pz[:, 0]
    idx0 = jnp.zeros((Bb, SP), jnp.int32)
    sx0 = jnp.where(iota_s == 0, lx0[:, None], F32(0.0))
    sy0 = jnp.where(iota_s == 0, ly0[:, None], F32(0.0))
    sz0 = jnp.where(iota_s == 0, lz0[:, None], F32(0.0))

    def body(i, carry):
        dists, lx, ly, lz, idx, sx, sy, sz = carry
        d = ((px - lx[:, None]) ** 2 + (py - ly[:, None]) ** 2
             + (pz - lz[:, None]) ** 2)
        dists = jnp.minimum(dists, d)
        nxt = jnp.argmax(dists, axis=-1).astype(jnp.int32)
        oh = iota_p == nxt[:, None]
        nlx = jnp.max(jnp.where(oh, px, -BIG), axis=-1)
        nly = jnp.max(jnp.where(oh, py, -BIG), axis=-1)
        nlz = jnp.max(jnp.where(oh, pz, -BIG), axis=-1)
        sel = iota_s == i
        idx = jnp.where(sel, nxt[:, None], idx)
        sx = jnp.where(sel, nlx[:, None], sx)
        sy = jnp.where(sel, nly[:, None], sy)
        sz = jnp.where(sel, nlz[:, None], sz)
        return dists, nlx, nly, nlz, idx, sx, sy, sz

    carry = lax.fori_loop(
        1, nsamp, body, (dists0, lx0, ly0, lz0, idx0, sx0, sy0, sz0))
    _, _, _, _, idx, sx, sy, sz = carry
    idx_ref[...] = idx.astype(jnp.int32)
    sx_ref[...] = sx
    sy_ref[...] = sy
    sz_ref[...] = sz


def _fps(px, py, pz, n_real, nsamp, SP):
    Bb, Pp = px.shape
    out_shape = (
        jax.ShapeDtypeStruct((Bb, SP), jnp.int32),
        jax.ShapeDtypeStruct((Bb, SP), F32),
        jax.ShapeDtypeStruct((Bb, SP), F32),
        jax.ShapeDtypeStruct((Bb, SP), F32),
    )
    return pl.pallas_call(
        functools.partial(_fps_body, n_real, nsamp),
        out_shape=out_shape,
    )(px, py, pz)


def _ball_topk_body(r2, n_real, stride, cx_ref, cy_ref, cz_ref,
                    px_ref, py_ref, pz_ref, nidx_ref, vc_ref):
    b = pl.program_id(0)
    SP = cx_ref.shape[2]
    Pp = px_ref.shape[2]
    cx = cx_ref[0, 0, :]
    cy = cy_ref[0, 0, :]
    cz = cz_ref[0, 0, :]
    px = px_ref[0, 0, :]
    py = py_ref[0, 0, :]
    pz = pz_ref[0, 0, :]
    d = ((cx[:, None] - px[None, :]) ** 2 + (cy[:, None] - py[None, :]) ** 2
         + (cz[:, None] - pz[None, :]) ** 2)
    iota_p = lax.broadcasted_iota(jnp.int32, (SP, Pp), 1)
    dm = jnp.where((d <= r2) & (iota_p < n_real), d, BIG)
    cnt = jnp.sum((dm < F32(5e9)).astype(jnp.int32), axis=-1)
    iota_k = lax.broadcasted_iota(jnp.int32, (SP, K), 1)
    nacc = jnp.zeros((SP, K), jnp.int32)
    for k in range(K):
        sel = jnp.argmin(dm, axis=-1).astype(jnp.int32)
        nacc = jnp.where(iota_k == k, sel[:, None], nacc)
        dm = jnp.where(iota_p == sel[:, None], BIG, dm)
    nidx_ref[0] = (nacc + b * stride).astype(jnp.int32)
    vc_ref[0, 0] = jnp.minimum(cnt, K)


def _ball_topk(cx, cy, cz, px, py, pz, r2, n_real, stride):
    Bb, SP = cx.shape
    Pp = px.shape[1]
    cx3 = cx.reshape(Bb, 1, SP)
    cy3 = cy.reshape(Bb, 1, SP)
    cz3 = cz.reshape(Bb, 1, SP)
    px3 = px.reshape(Bb, 1, Pp)
    py3 = py.reshape(Bb, 1, Pp)
    pz3 = pz.reshape(Bb, 1, Pp)
    spec_c = pl.BlockSpec((1, 1, SP), lambda b: (b, 0, 0))
    spec_p = pl.BlockSpec((1, 1, Pp), lambda b: (b, 0, 0))
    out_shape = (
        jax.ShapeDtypeStruct((Bb, SP, K), jnp.int32),
        jax.ShapeDtypeStruct((Bb, 1, SP), jnp.int32),
    )
    return pl.pallas_call(
        functools.partial(_ball_topk_body, r2, n_real, stride),
        grid=(Bb,),
        in_specs=[spec_c, spec_c, spec_c, spec_p, spec_p, spec_p],
        out_specs=(pl.BlockSpec((1, SP, K), lambda b: (b, 0, 0)),
                   pl.BlockSpec((1, 1, SP), lambda b: (b, 0, 0))),
        out_shape=out_shape,
    )(cx3, cy3, cz3, px3, py3, pz3)


def _gather_rows(table, idx):
    M = idx.shape[0]
    Dp = table.shape[1]
    info = plsc.get_sparse_core_info()
    NC, NS = info.num_cores, info.num_subcores
    NW = NC * NS
    assert M % (NW * 128) == 0
    CPW = M // (NW * 128)
    mesh = plsc.VectorSubcoreMesh(core_axis_name="c", subcore_axis_name="s")

    G = 1
    for cand in (2, 4, 8):
        if CPW % cand == 0 and cand * 128 * Dp * 4 <= 380 * 1024:
            G = cand
    NG = CPW // G

    @functools.partial(
        pl.kernel,
        out_type=jax.ShapeDtypeStruct((M, Dp), F32),
        mesh=mesh,
        compiler_params=pltpu.CompilerParams(use_tc_tiling_on_sc=False),
        scratch_types=[
            pltpu.VMEM((CPW * 128,), jnp.int32),
            pltpu.VMEM((G * 128, Dp), F32),
            pltpu.SemaphoreType.DMA,
        ],
    )
    def gk(idx_hbm, table_hbm, out_hbm, idx_v, rows_v, sem):
        wid = lax.axis_index("s") * NC + lax.axis_index("c")
        base = wid * (CPW * 128)
        pltpu.sync_copy(idx_hbm.at[pl.ds(base, CPW * 128)], idx_v)

        def body(g, _):
            descs = []
            for b in range(G):
                descs.append(pltpu.async_copy(
                    table_hbm.at[idx_v.at[pl.ds((g * G + b) * 128, 128)]],
                    rows_v.at[pl.ds(b * 128, 128)], sem))
            for dsc in descs:
                dsc.wait()
            pltpu.sync_copy(rows_v,
                            out_hbm.at[pl.ds(base + g * (G * 128), G * 128)])
            return 0

        lax.fori_loop(0, NG, body, 0)

    return gk(idx, table)


def _sa_mlp_body(SBLK, g_ref, c_ref, vc_ref, w1_ref, b1_ref, w2_ref, b2_ref,
                 w3_ref, b3_ref, out_ref):
    Dp = g_ref.shape[-1]
    cp = c_ref[0]
    g = (g_ref[0].reshape(SBLK, K, Dp) - cp[:, None, :]).reshape(SBLK * K, Dp)
    h = jnp.maximum(jnp.dot(g, w1_ref[...],
                            preferred_element_type=F32) + b1_ref[0], 0.0)
    h = jnp.maximum(jnp.dot(h, w2_ref[...],
                            preferred_element_type=F32) + b2_ref[0], 0.0)
    h = jnp.maximum(jnp.dot(h, w3_ref[...],
                            preferred_element_type=F32) + b3_ref[0], 0.0)
    C = h.shape[-1]
    h3 = h.reshape(SBLK, K, C)
    iota_k = lax.broadcasted_iota(jnp.int32, (SBLK, K), 1)
    vmf = (iota_k < vc_ref[0, 0][:, None]).astype(F32)
    h3 = jnp.where(vmf[:, :, None] > 0, h3, -BIG)
    out_ref[0] = jnp.max(h3, axis=1)


def _sa_mlp_max(g, cpad, vc, w1, b1, w2, b2, w3, b3, SP, SBLK):
    Dp = g.shape[-1]
    C = w3.shape[1]
    nblk = SP // SBLK
    grid = (B, nblk)
    spec_g = pl.BlockSpec((1, SBLK * K, Dp), lambda b, s: (b, s, 0))
    spec_c = pl.BlockSpec((1, SBLK, Dp), lambda b, s: (b, s, 0))
    spec_vc = pl.BlockSpec((1, 1, SBLK), lambda b, s: (b, 0, s))
    spec_w = lambda sh: pl.BlockSpec(sh, lambda b, s: (0, 0))
    out_spec = pl.BlockSpec((1, SBLK, C), lambda b, s: (b, s, 0))
    return pl.pallas_call(
        functools.partial(_sa_mlp_body, SBLK),
        grid=grid,
        in_specs=[spec_g, spec_c, spec_vc,
                  spec_w(w1.shape), spec_w((1, w1.shape[1])),
                  spec_w(w2.shape), spec_w((1, w2.shape[1])),
                  spec_w(w3.shape), spec_w((1, w3.shape[1]))],
        out_specs=out_spec,
        out_shape=jax.ShapeDtypeStruct((B, SP, C), F32),
    )(g, cpad, vc, w1, b1.reshape(1, -1), w2, b2.reshape(1, -1),
      w3, b3.reshape(1, -1))


def _sa3fp3_body(x2_ref, p2x_ref, p2y_ref, p2z_ref,
                 w1a_ref, w1b_ref, b1_ref, w2_ref, b2_ref, w3_ref, b3_ref,
                 fw1a_ref, fw1b_ref, fb1_ref, fw2_ref, fb2_ref, out_ref):
    x2 = x2_ref[0]
    pcat = jnp.concatenate(
        [p2x_ref[0, 0, :][:, None], p2y_ref[0, 0, :][:, None],
         p2z_ref[0, 0, :][:, None]], axis=-1)
    h = jnp.dot(x2, w1a_ref[...], preferred_element_type=F32)
    h = h + jnp.dot(pcat, w1b_ref[...], preferred_element_type=F32)
    h = jnp.maximum(h + b1_ref[0], 0.0)
    h = jnp.maximum(jnp.dot(h, w2_ref[...],
                            preferred_element_type=F32) + b2_ref[0], 0.0)
    h = jnp.maximum(jnp.dot(h, w3_ref[...],
                            preferred_element_type=F32) + b3_ref[0], 0.0)
    rows = lax.broadcasted_iota(jnp.int32, h.shape, 0)
    h = jnp.where(rows < S2, h, -BIG)
    g = jnp.max(h, axis=0, keepdims=True)
    t = jnp.dot(g, fw1a_ref[...], preferred_element_type=F32)
    t = t + jnp.dot(x2, fw1b_ref[...], preferred_element_type=F32)
    t = jnp.maximum(t + fb1_ref[0], 0.0)
    f3 = jnp.maximum(jnp.dot(t, fw2_ref[...],
                             preferred_element_type=F32) + fb2_ref[0], 0.0)
    out_ref[0] = f3


def _sa3fp3(x2, p2x, p2y, p2z, sa3, fp3):
    (w1, b1), (w2, b2), (w3, b3) = sa3
    (fw1, fb1), (fw2, fb2) = fp3
    w1a, w1b = w1[:256], w1[256:259]
    fw1a, fw1b = fw1[:1024], fw1[1024:1280]
    spec_x = pl.BlockSpec((1, SP2, 256), lambda b: (b, 0, 0))
    spec_p = pl.BlockSpec((1, 1, SP2), lambda b: (b, 0, 0))
    spec_w = lambda sh: pl.BlockSpec(sh, lambda b: (0, 0))
    p3 = lambda a: a.reshape(B, 1, SP2)
    return pl.pallas_call(
        _sa3fp3_body,
        grid=(B,),
        in_specs=[spec_x, spec_p, spec_p, spec_p,
                  spec_w(w1a.shape), spec_w(w1b.shape), spec_w((1, 256)),
                  spec_w(w2.shape), spec_w((1, 512)),
                  spec_w(w3.shape), spec_w((1, 1024)),
                  spec_w(fw1a.shape), spec_w(fw1b.shape), spec_w((1, 256)),
                  spec_w(fw2.shape), spec_w((1, 256))],
        out_specs=pl.BlockSpec((1, SP2, 256), lambda b: (b, 0, 0)),
        out_shape=jax.ShapeDtypeStruct((B, SP2, 256), F32),
    )(x2, p3(p2x), p3(p2y), p3(p2z),
      w1a, w1b, b1.reshape(1, -1), w2, b2.reshape(1, -1),
      w3, b3.reshape(1, -1), fw1a, fw1b, fb1.reshape(1, -1),
      fw2, fb2.reshape(1, -1))


def _knn3_weight_matrix(tx, ty, tz, sx, sy, sz, n_src):
    T = tx.shape[0]
    S = sx.shape[0]
    d = ((tx[:, None] - sx[None, :]) ** 2 + (ty[:, None] - sy[None, :]) ** 2
         + (tz[:, None] - sz[None, :]) ** 2)
    iota_c = lax.broadcasted_iota(jnp.int32, (T, S), 1)
    d = jnp.where(iota_c < n_src, d, BIG)
    wmat = jnp.zeros((T, S), F32)
    wsum = jnp.zeros((T,), F32)
    for _ in range(3):
        sel = jnp.argmin(d, axis=-1).astype(jnp.int32)
        rmin = jnp.min(d, axis=-1)
        w = 1.0 / (jnp.maximum(rmin, 0.0) + F32(1e-8))
        e = iota_c == sel[:, None]
        wmat = wmat + jnp.where(e, w[:, None], 0.0)
        wsum = wsum + w
        d = jnp.where(e, BIG, d)
    return wmat, wsum


def _interp_fp2_body(p1x_ref, p1y_ref, p1z_ref, p2x_ref, p2y_ref, p2z_ref,
                     f3_ref, x1_ref, w1a_ref, w1b_ref, b1_ref,
                     w2_ref, b2_ref, out_ref):
    wmat, wsum = _knn3_weight_matrix(
        p1x_ref[0, 0, :], p1y_ref[0, 0, :], p1z_ref[0, 0, :],
        p2x_ref[0, 0, :], p2y_ref[0, 0, :], p2z_ref[0, 0, :], S2)
    interp = jnp.dot(wmat, f3_ref[0],
                     preferred_element_type=F32) / wsum[:, None]
    h = jnp.dot(interp, w1a_ref[...], preferred_element_type=F32)
    h = h + jnp.dot(x1_ref[0], w1b_ref[...], preferred_element_type=F32)
    h = jnp.maximum(h + b1_ref[0], 0.0)
    f2 = jnp.maximum(jnp.dot(h, w2_ref[...],
                             preferred_element_type=F32) + b2_ref[0], 0.0)
    out_ref[0] = f2


def _interp_fp2(p1x, p1y, p1z, p2x, p2y, p2z, f3, x1, fp2):
    (w1, b1), (w2, b2) = fp2
    w1a, w1b = w1[:256], w1[256:384]
    spec_p1 = pl.BlockSpec((1, 1, SP1), lambda b: (b, 0, 0))
    spec_p2 = pl.BlockSpec((1, 1, SP2), lambda b: (b, 0, 0))
    spec_w = lambda sh: pl.BlockSpec(sh, lambda b: (0, 0))
    r1 = lambda a: a.reshape(B, 1, SP1)
    r2_ = lambda a: a.reshape(B, 1, SP2)
    return pl.pallas_call(
        _interp_fp2_body,
        grid=(B,),
        in_specs=[spec_p1, spec_p1, spec_p1, spec_p2, spec_p2, spec_p2,
                  pl.BlockSpec((1, SP2, 256), lambda b: (b, 0, 0)),
                  pl.BlockSpec((1, SP1, 128), lambda b: (b, 0, 0)),
                  spec_w(w1a.shape), spec_w(w1b.shape), spec_w((1, 256)),
                  spec_w(w2.shape), spec_w((1, 128))],
        out_specs=pl.BlockSpec((1, SP1, 128), lambda b: (b, 0, 0)),
        out_shape=jax.ShapeDtypeStruct((B, SP1, 128), F32),
    )(r1(p1x), r1(p1y), r1(p1z), r2_(p2x), r2_(p2y), r2_(p2z), f3, x1,
      w1a, w1b, b1.reshape(1, -1), w2, b2.reshape(1, -1))


def _interp_fp1_head_body(pbx_ref, pby_ref, pbz_ref, p1x_ref, p1y_ref,
                          p1z_ref, f2_ref, xb_ref,
                          w1a_ref, w1b_ref, b1_ref, w2_ref, b2_ref,
                          w3_ref, b3_ref, hw1_ref, hb1_ref, hw2_ref,
                          hb2_ref, hw3_ref, hb3_ref, out_ref):
    wmat, wsum = _knn3_weight_matrix(
        pbx_ref[0, 0, :], pby_ref[0, 0, :], pbz_ref[0, 0, :],
        p1x_ref[0, 0, :], p1y_ref[0, 0, :], p1z_ref[0, 0, :], S1)
    interp = jnp.dot(wmat, f2_ref[0],
                     preferred_element_type=F32) / wsum[:, None]
    h = jnp.dot(interp, w1a_ref[...], preferred_element_type=F32)
    h = h + jnp.dot(xb_ref[0], w1b_ref[...], preferred_element_type=F32)
    h = jnp.maximum(h + b1_ref[0], 0.0)
    h = jnp.maximum(jnp.dot(h, w2_ref[...],
                            preferred_element_type=F32) + b2_ref[0], 0.0)
    h = jnp.maximum(jnp.dot(h, w3_ref[...],
                            preferred_element_type=F32) + b3_ref[0], 0.0)
    h = jnp.maximum(jnp.dot(h, hw1_ref[...],
                            preferred_element_type=F32) + hb1_ref[0], 0.0)
    h = jnp.maximum(jnp.dot(h, hw2_ref[...],
                            preferred_element_type=F32) + hb2_ref[0], 0.0)
    h = jnp.dot(h, hw3_ref[...], preferred_element_type=F32) + hb3_ref[0]
    out_ref[0] = jnp.tanh(h)


def _interp_fp1_head(pbx, pby, pbz, p1x, p1y, p1z, f2, xb, fp1, head):
    (w1, b1), (w2, b2), (w3, b3) = fp1
    (hw1, hb1), (hw2, hb2), (hw3, hb3) = head
    w1a, w1b = w1[:128], w1[128:131]
    spec_pb = pl.BlockSpec((1, 1, P), lambda b: (b, 0, 0))
    spec_p1 = pl.BlockSpec((1, 1, SP1), lambda b: (b, 0, 0))
    spec_w = lambda sh: pl.BlockSpec(sh, lambda b: (0, 0))
    rb = lambda a: a.reshape(B, 1, P)
    r1 = lambda a: a.reshape(B, 1, SP1)
    return pl.pallas_call(
        _interp_fp1_head_body,
        grid=(B,),
        in_specs=[spec_pb, spec_pb, spec_pb, spec_p1, spec_p1, spec_p1,
                  pl.BlockSpec((1, SP1, 128), lambda b: (b, 0, 0)),
                  pl.BlockSpec((1, P, 3), lambda b: (b, 0, 0)),
                  spec_w(w1a.shape), spec_w(w1b.shape), spec_w((1, 128)),
                  spec_w(w2.shape), spec_w((1, 128)),
                  spec_w(w3.shape), spec_w((1, 128)),
                  spec_w(hw1.shape), spec_w((1, 512)),
                  spec_w(hw2.shape), spec_w((1, 256)),
                  spec_w(hw3.shape), spec_w((1, 32))],
        out_specs=pl.BlockSpec((1, P, 32), lambda b: (b, 0, 0)),
        out_shape=jax.ShapeDtypeStruct((B, P, 32), F32),
    )(rb(pbx), rb(pby), rb(pbz), r1(p1x), r1(p1y), r1(p1z), f2, xb,
      w1a, w1b, b1.reshape(1, -1), w2, b2.reshape(1, -1),
      w3, b3.reshape(1, -1), hw1, hb1.reshape(1, -1),
      hw2, hb2.reshape(1, -1), hw3, hb3.reshape(1, -1))


def _pad_rows(w, rows):
    out = jnp.zeros((rows, w.shape[1]), F32)
    return out.at[: w.shape[0]].set(w)


def kernel(x, pos, batch, sa1, sa2, sa3, fp3, fp2, fp1, head):
    xb = x.reshape(B, P, 3)
    pb = pos.reshape(B, P, 3)
    pbx, pby, pbz = pb[..., 0], pb[..., 1], pb[..., 2]

    idx1, s1x, s1y, s1z = _fps(pbx, pby, pbz, P, S1, SP1)
    nidx1, vc1 = _ball_topk(s1x, s1y, s1z, pbx, pby, pbz,
                            r2=np.float32(0.2 * 0.2), n_real=P, stride=P)
    table1 = jnp.concatenate(
        [x, pos, jnp.zeros((N, 10), F32)], axis=1)
    g1 = _gather_rows(table1, nidx1.reshape(-1))
    cpad1 = jnp.concatenate(
        [jnp.zeros((B, SP1, 3), F32), s1x[..., None], s1y[..., None],
         s1z[..., None], jnp.zeros((B, SP1, 10), F32)], axis=-1)
    (w11, b11), (w12, b12), (w13, b13) = sa1
    x1 = _sa_mlp_max(g1.reshape(B, SP1 * K, 16), cpad1, vc1,
                     _pad_rows(w11, 16), b11, w12, b12, w13, b13,
                     SP=SP1, SBLK=128)

    idx2, s2x, s2y, s2z = _fps(s1x, s1y, s1z, S1, S2, SP2)
    nidx2, vc2 = _ball_topk(s2x, s2y, s2z, s1x, s1y, s1z,
                            r2=np.float32(0.4 * 0.4), n_real=S1, stride=SP1)
    table2 = jnp.concatenate(
        [x1.reshape(B * SP1, 128), s1x.reshape(B * SP1, 1),
         s1y.reshape(B * SP1, 1), s1z.reshape(B * SP1, 1),
         jnp.zeros((B * SP1, 13), F32)], axis=1)
    g2 = _gather_rows(table2, nidx2.reshape(-1))
    cpad2 = jnp.concatenate(
        [jnp.zeros((B, SP2, 128), F32), s2x[..., None], s2y[..., None],
         s2z[..., None], jnp.zeros((B, SP2, 13), F32)], axis=-1)
    (w21, b21), (w22, b22), (w23, b23) = sa2
    x2 = _sa_mlp_max(g2.reshape(B, SP2 * K, 144), cpad2, vc2,
                     _pad_rows(w21, 144), b21, w22, b22, w23, b23,
                     SP=SP2, SBLK=128)

    f3 = _sa3fp3(x2, s2x, s2y, s2z, sa3, fp3)
    f2 = _interp_fp2(s1x, s1y, s1z, s2x, s2y, s2z, f3, x1, fp2)
    outb = _interp_fp1_head(pbx, pby, pbz, s1x, s1y, s1z, f2, xb, fp1, head)

    out = outb.reshape(N, 32)
    gidx = (idx1[:, :S1]
            + jnp.arange(B, dtype=jnp.int32)[:, None] * P).reshape(-1)
    return out, gidx


# speedup vs baseline: 14.4047x; 1.0286x over previous
import functools

import jax
import jax.numpy as jnp
import numpy as np
from jax import lax
from jax.experimental import pallas as pl
from jax.experimental.pallas import tpu as pltpu
from jax.experimental.pallas import tpu_sc as plsc

B = 16
P = 2048
N = B * P
K = 32
S1 = 409
SP1 = 512
S2 = 102
SP2 = 128
BIG = np.float32(1e10)
F32 = np.float32


def _fps_body(n_real, nsamp, px_ref, py_ref, pz_ref,
              idx_ref, sx_ref, sy_ref, sz_ref):
    Bb, Pp = px_ref.shape
    SP = idx_ref.shape[1]
    px = px_ref[...]
    py = py_ref[...]
    pz = pz_ref[...]
    iota_p = lax.broadcasted_iota(jnp.int32, (Bb, Pp), 1)
    iota_s = lax.broadcasted_iota(jnp.int32, (Bb, SP), 1)
    dists0 = jnp.where(iota_p < n_real, BIG, F32(-1.0))
    lx0 = px[:, 0]
    ly0 = py[:, 0]
    lz0 = pz[:, 0]
    idx0 = jnp.zeros((Bb, SP), jnp.int32)
    sx0 = jnp.where(iota_s == 0, lx0[:, None], F32(0.0))
    sy0 = jnp.where(iota_s == 0, ly0[:, None], F32(0.0))
    sz0 = jnp.where(iota_s == 0, lz0[:, None], F32(0.0))

    def body(i, carry):
        dists, lx, ly, lz, idx, sx, sy, sz = carry
        d = ((px - lx[:, None]) ** 2 + (py - ly[:, None]) ** 2
             + (pz - lz[:, None]) ** 2)
        dists = jnp.minimum(dists, d)
        nxt = jnp.argmax(dists, axis=-1).astype(jnp.int32)
        oh = iota_p == nxt[:, None]
        nlx = jnp.max(jnp.where(oh, px, -BIG), axis=-1)
        nly = jnp.max(jnp.where(oh, py, -BIG), axis=-1)
        nlz = jnp.max(jnp.where(oh, pz, -BIG), axis=-1)
        sel = iota_s == i
        idx = jnp.where(sel, nxt[:, None], idx)
        sx = jnp.where(sel, nlx[:, None], sx)
        sy = jnp.where(sel, nly[:, None], sy)
        sz = jnp.where(sel, nlz[:, None], sz)
        return dists, nlx, nly, nlz, idx, sx, sy, sz

    carry = lax.fori_loop(
        1, nsamp, body, (dists0, lx0, ly0, lz0, idx0, sx0, sy0, sz0))
    _, _, _, _, idx, sx, sy, sz = carry
    idx_ref[...] = idx.astype(jnp.int32)
    sx_ref[...] = sx
    sy_ref[...] = sy
    sz_ref[...] = sz


def _fps(px, py, pz, n_real, nsamp, SP):
    Bb, Pp = px.shape
    out_shape = (
        jax.ShapeDtypeStruct((Bb, SP), jnp.int32),
        jax.ShapeDtypeStruct((Bb, SP), F32),
        jax.ShapeDtypeStruct((Bb, SP), F32),
        jax.ShapeDtypeStruct((Bb, SP), F32),
    )
    return pl.pallas_call(
        functools.partial(_fps_body, n_real, nsamp),
        out_shape=out_shape,
    )(px, py, pz)


def _ball_topk_body(r2, n_real, stride, cx_ref, cy_ref, cz_ref,
                    px_ref, py_ref, pz_ref, nidx_ref, vc_ref):
    b = pl.program_id(0)
    SP = cx_ref.shape[2]
    Pp = px_ref.shape[2]
    cx = cx_ref[0, 0, :]
    cy = cy_ref[0, 0, :]
    cz = cz_ref[0, 0, :]
    px = px_ref[0, 0, :]
    py = py_ref[0, 0, :]
    pz = pz_ref[0, 0, :]
    d = ((cx[:, None] - px[None, :]) ** 2 + (cy[:, None] - py[None, :]) ** 2
         + (cz[:, None] - pz[None, :]) ** 2)
    iota_p = lax.broadcasted_iota(jnp.int32, (SP, Pp), 1)
    dm = jnp.where((d <= r2) & (iota_p < n_real), d, BIG)
    cnt = jnp.sum((dm < F32(5e9)).astype(jnp.int32), axis=-1)
    iota_k = lax.broadcasted_iota(jnp.int32, (SP, K), 1)
    nacc = jnp.zeros((SP, K), jnp.int32)
    for k in range(K):
        sel = jnp.argmin(dm, axis=-1).astype(jnp.int32)
        nacc = jnp.where(iota_k == k, sel[:, None], nacc)
        dm = jnp.where(iota_p == sel[:, None], BIG, dm)
    nidx_ref[0] = (nacc + b * stride).astype(jnp.int32)
    vc_ref[0, 0] = jnp.minimum(cnt, K)


def _ball_topk(cx, cy, cz, px, py, pz, r2, n_real, stride):
    Bb, SP = cx.shape
    Pp = px.shape[1]
    cx3 = cx.reshape(Bb, 1, SP)
    cy3 = cy.reshape(Bb, 1, SP)
    cz3 = cz.reshape(Bb, 1, SP)
    px3 = px.reshape(Bb, 1, Pp)
    py3 = py.reshape(Bb, 1, Pp)
    pz3 = pz.reshape(Bb, 1, Pp)
    spec_c = pl.BlockSpec((1, 1, SP), lambda b: (b, 0, 0))
    spec_p = pl.BlockSpec((1, 1, Pp), lambda b: (b, 0, 0))
    out_shape = (
        jax.ShapeDtypeStruct((Bb, SP, K), jnp.int32),
        jax.ShapeDtypeStruct((Bb, 1, SP), jnp.int32),
    )
    return pl.pallas_call(
        functools.partial(_ball_topk_body, r2, n_real, stride),
        grid=(Bb,),
        in_specs=[spec_c, spec_c, spec_c, spec_p, spec_p, spec_p],
        out_specs=(pl.BlockSpec((1, SP, K), lambda b: (b, 0, 0)),
                   pl.BlockSpec((1, 1, SP), lambda b: (b, 0, 0))),
        out_shape=out_shape,
    )(cx3, cy3, cz3, px3, py3, pz3)


def _gather_rows(table, idx):
    M = idx.shape[0]
    Dp = table.shape[1]
    info = plsc.get_sparse_core_info()
    NC, NS = info.num_cores, info.num_subcores
    NW = NC * NS
    assert M % (NW * 128) == 0
    CPW = M // (NW * 128)
    mesh = plsc.VectorSubcoreMesh(core_axis_name="c", subcore_axis_name="s")

    G = 1
    for cand in (2, 4, 8):
        if CPW % cand == 0 and cand * 128 * Dp * 4 <= 380 * 1024:
            G = cand
    NG = CPW // G

    @functools.partial(
        pl.kernel,
        out_type=jax.ShapeDtypeStruct((M, Dp), F32),
        mesh=mesh,
        compiler_params=pltpu.CompilerParams(use_tc_tiling_on_sc=False),
        scratch_types=[
            pltpu.VMEM((CPW * 128,), jnp.int32),
            pltpu.VMEM((G * 128, Dp), F32),
            pltpu.SemaphoreType.DMA,
        ],
    )
    def gk(idx_hbm, table_hbm, out_hbm, idx_v, rows_v, sem):
        wid = lax.axis_index("s") * NC + lax.axis_index("c")
        base = wid * (CPW * 128)
        pltpu.sync_copy(idx_hbm.at[pl.ds(base, CPW * 128)], idx_v)

        def body(g, _):
            descs = []
            for b in range(G):
                descs.append(pltpu.async_copy(
                    table_hbm.at[idx_v.at[pl.ds((g * G + b) * 128, 128)]],
                    rows_v.at[pl.ds(b * 128, 128)], sem))
            for dsc in descs:
                dsc.wait()
            pltpu.sync_copy(rows_v,
                            out_hbm.at[pl.ds(base + g * (G * 128), G * 128)])
            return 0

        lax.fori_loop(0, NG, body, 0)

    return gk(idx, table)


def _sa_mlp_body(SBLK, g_ref, c_ref, vc_ref, w1_ref, b1_ref, w2_ref, b2_ref,
                 w3_ref, b3_ref, out_ref):
    Dp = g_ref.shape[-1]
    cp = c_ref[0]
    g = (g_ref[0].reshape(SBLK, K, Dp) - cp[:, None, :]).reshape(SBLK * K, Dp)
    h = jnp.maximum(jnp.dot(g, w1_ref[...],
                            preferred_element_type=F32) + b1_ref[0], 0.0)
    h = jnp.maximum(jnp.dot(h, w2_ref[...],
                            preferred_element_type=F32) + b2_ref[0], 0.0)
    h = jnp.maximum(jnp.dot(h, w3_ref[...],
                            preferred_element_type=F32) + b3_ref[0], 0.0)
    C = h.shape[-1]
    h3 = h.reshape(SBLK, K, C)
    iota_k = lax.broadcasted_iota(jnp.int32, (SBLK, K), 1)
    vmf = (iota_k < vc_ref[0, 0][:, None]).astype(F32)
    h3 = jnp.where(vmf[:, :, None] > 0, h3, -BIG)
    out_ref[0] = jnp.max(h3, axis=1)


def _sa_mlp_max(g, cpad, vc, w1, b1, w2, b2, w3, b3, SP, SBLK):
    Dp = g.shape[-1]
    C = w3.shape[1]
    nblk = SP // SBLK
    grid = (B, nblk)
    spec_g = pl.BlockSpec((1, SBLK * K, Dp), lambda b, s: (b, s, 0))
    spec_c = pl.BlockSpec((1, SBLK, Dp), lambda b, s: (b, s, 0))
    spec_vc = pl.BlockSpec((1, 1, SBLK), lambda b, s: (b, 0, s))
    spec_w = lambda sh: pl.BlockSpec(sh, lambda b, s: (0, 0))
    out_spec = pl.BlockSpec((1, SBLK, C), lambda b, s: (b, s, 0))
    return pl.pallas_call(
        functools.partial(_sa_mlp_body, SBLK),
        grid=grid,
        in_specs=[spec_g, spec_c, spec_vc,
                  spec_w(w1.shape), spec_w((1, w1.shape[1])),
                  spec_w(w2.shape), spec_w((1, w2.shape[1])),
                  spec_w(w3.shape), spec_w((1, w3.shape[1]))],
        out_specs=out_spec,
        out_shape=jax.ShapeDtypeStruct((B, SP, C), F32),
    )(g, cpad, vc, w1, b1.reshape(1, -1), w2, b2.reshape(1, -1),
      w3, b3.reshape(1, -1))


def _knn3_weight_matrix(tx, ty, tz, sx, sy, sz, n_src):
    T = tx.shape[0]
    S = sx.shape[0]
    d = ((tx[:, None] - sx[None, :]) ** 2 + (ty[:, None] - sy[None, :]) ** 2
         + (tz[:, None] - sz[None, :]) ** 2)
    iota_c = lax.broadcasted_iota(jnp.int32, (T, S), 1)
    d = jnp.where(iota_c < n_src, d, BIG)
    wmat = jnp.zeros((T, S), F32)
    wsum = jnp.zeros((T,), F32)
    for _ in range(3):
        sel = jnp.argmin(d, axis=-1).astype(jnp.int32)
        rmin = jnp.min(d, axis=-1)
        w = 1.0 / (jnp.maximum(rmin, 0.0) + F32(1e-8))
        e = iota_c == sel[:, None]
        wmat = wmat + jnp.where(e, w[:, None], 0.0)
        wsum = wsum + w
        d = jnp.where(e, BIG, d)
    return wmat, wsum


def _tail_body(x2_ref, p2x_ref, p2y_ref, p2z_ref, p1x_ref, p1y_ref, p1z_ref,
               pbx_ref, pby_ref, pbz_ref, x1_ref, xb_ref,
               s31a_ref, s31b_ref, s3b1_ref, s32_ref, s3b2_ref,
               s33_ref, s3b3_ref,
               f31a_ref, f31b_ref, f3b1_ref, f32_ref, f3b2_ref,
               g1a_ref, g1b_ref, gb1_ref, g2_ref, gb2_ref,
               q1a_ref, q1b_ref, qb1_ref, q2_ref, qb2_ref, q3_ref, qb3_ref,
               hw1_ref, hb1_ref, hw2_ref, hb2_ref, hw3_ref, hb3_ref,
               out_ref):
    x2 = x2_ref[0]
    pcat = jnp.concatenate(
        [p2x_ref[0, 0, :][:, None], p2y_ref[0, 0, :][:, None],
         p2z_ref[0, 0, :][:, None]], axis=-1)
    h = jnp.dot(x2, s31a_ref[...], preferred_element_type=F32)
    h = h + jnp.dot(pcat, s31b_ref[...], preferred_element_type=F32)
    h = jnp.maximum(h + s3b1_ref[0], 0.0)
    h = jnp.maximum(jnp.dot(h, s32_ref[...],
                            preferred_element_type=F32) + s3b2_ref[0], 0.0)
    h = jnp.maximum(jnp.dot(h, s33_ref[...],
                            preferred_element_type=F32) + s3b3_ref[0], 0.0)
    rows = lax.broadcasted_iota(jnp.int32, h.shape, 0)
    h = jnp.where(rows < S2, h, -BIG)
    g = jnp.max(h, axis=0, keepdims=True)
    t = jnp.dot(g, f31a_ref[...], preferred_element_type=F32)
    t = t + jnp.dot(x2, f31b_ref[...], preferred_element_type=F32)
    t = jnp.maximum(t + f3b1_ref[0], 0.0)
    f3 = jnp.maximum(jnp.dot(t, f32_ref[...],
                             preferred_element_type=F32) + f3b2_ref[0], 0.0)
    wmat2, wsum2 = _knn3_weight_matrix(
        p1x_ref[0, 0, :], p1y_ref[0, 0, :], p1z_ref[0, 0, :],
        p2x_ref[0, 0, :], p2y_ref[0, 0, :], p2z_ref[0, 0, :], S2)
    interp2 = jnp.dot(wmat2, f3,
                      preferred_element_type=F32) / wsum2[:, None]
    h = jnp.dot(interp2, g1a_ref[...], preferred_element_type=F32)
    h = h + jnp.dot(x1_ref[0], g1b_ref[...], preferred_element_type=F32)
    h = jnp.maximum(h + gb1_ref[0], 0.0)
    f2 = jnp.maximum(jnp.dot(h, g2_ref[...],
                             preferred_element_type=F32) + gb2_ref[0], 0.0)
    wmat1, wsum1 = _knn3_weight_matrix(
        pbx_ref[0, 0, :], pby_ref[0, 0, :], pbz_ref[0, 0, :],
        p1x_ref[0, 0, :], p1y_ref[0, 0, :], p1z_ref[0, 0, :], S1)
    interp1 = jnp.dot(wmat1, f2,
                      preferred_element_type=F32) / wsum1[:, None]
    h = jnp.dot(interp1, q1a_ref[...], preferred_element_type=F32)
    h = h + jnp.dot(xb_ref[0], q1b_ref[...], preferred_element_type=F32)
    h = jnp.maximum(h + qb1_ref[0], 0.0)
    h = jnp.maximum(jnp.dot(h, q2_ref[...],
                            preferred_element_type=F32) + qb2_ref[0], 0.0)
    h = jnp.maximum(jnp.dot(h, q3_ref[...],
                            preferred_element_type=F32) + qb3_ref[0], 0.0)
    h = jnp.maximum(jnp.dot(h, hw1_ref[...],
                            preferred_element_type=F32) + hb1_ref[0], 0.0)
    h = jnp.maximum(jnp.dot(h, hw2_ref[...],
                            preferred_element_type=F32) + hb2_ref[0], 0.0)
    h = jnp.dot(h, hw3_ref[...], preferred_element_type=F32) + hb3_ref[0]
    out_ref[0] = jnp.tanh(h)


def _tail(x2, s2x, s2y, s2z, s1x, s1y, s1z, pbx, pby, pbz, x1, xb,
          sa3, fp3, fp2, fp1, head):
    (w1, b1), (w2, b2), (w3, b3) = sa3
    (fw1, fb1), (fw2, fb2) = fp3
    (gw1, gb1), (gw2, gb2) = fp2
    (qw1, qb1), (qw2, qb2), (qw3, qb3) = fp1
    (hw1, hb1), (hw2, hb2), (hw3, hb3) = head
    s31a, s31b = w1[:256], w1[256:259]
    f31a, f31b = fw1[:1024], fw1[1024:1280]
    g1a, g1b = gw1[:256], gw1[256:384]
    q1a, q1b = qw1[:128], qw1[128:131]
    spec_p2 = pl.BlockSpec((1, 1, SP2), lambda b: (b, 0, 0))
    spec_p1 = pl.BlockSpec((1, 1, SP1), lambda b: (b, 0, 0))
    spec_pb = pl.BlockSpec((1, 1, P), lambda b: (b, 0, 0))
    spec_w = lambda a: pl.BlockSpec(a.shape, lambda b: (0, 0))
    rv = lambda a: a.reshape(1, -1)
    r2_ = lambda a: a.reshape(B, 1, SP2)
    r1 = lambda a: a.reshape(B, 1, SP1)
    rb = lambda a: a.reshape(B, 1, P)
    ws = [s31a, s31b, rv(b1), w2, rv(b2), w3, rv(b3),
          f31a, f31b, rv(fb1), fw2, rv(fb2),
          g1a, g1b, rv(gb1), gw2, rv(gb2),
          q1a, q1b, rv(qb1), qw2, rv(qb2), qw3, rv(qb3),
          hw1, rv(hb1), hw2, rv(hb2), hw3, rv(hb3)]
    return pl.pallas_call(
        _tail_body,
        grid=(B,),
        in_specs=[pl.BlockSpec((1, SP2, 256), lambda b: (b, 0, 0)),
                  spec_p2, spec_p2, spec_p2, spec_p1, spec_p1, spec_p1,
                  spec_pb, spec_pb, spec_pb,
                  pl.BlockSpec((1, SP1, 128), lambda b: (b, 0, 0)),
                  pl.BlockSpec((1, P, 3), lambda b: (b, 0, 0))]
                 + [spec_w(a) for a in ws],
        out_specs=pl.BlockSpec((1, P, 32), lambda b: (b, 0, 0)),
        out_shape=jax.ShapeDtypeStruct((B, P, 32), F32),
    )(x2, r2_(s2x), r2_(s2y), r2_(s2z), r1(s1x), r1(s1y), r1(s1z),
      rb(pbx), rb(pby), rb(pbz), x1, xb, *ws)


def _pad_rows(w, rows):
    out = jnp.zeros((rows, w.shape[1]), F32)
    return out.at[: w.shape[0]].set(w)


def kernel(x, pos, batch, sa1, sa2, sa3, fp3, fp2, fp1, head):
    xb = x.reshape(B, P, 3)
    pb = pos.reshape(B, P, 3)
    pbx, pby, pbz = pb[..., 0], pb[..., 1], pb[..., 2]

    idx1, s1x, s1y, s1z = _fps(pbx, pby, pbz, P, S1, SP1)
    nidx1, vc1 = _ball_topk(s1x, s1y, s1z, pbx, pby, pbz,
                            r2=np.float32(0.2 * 0.2), n_real=P, stride=P)
    table1 = jnp.concatenate(
        [x, pos, jnp.zeros((N, 10), F32)], axis=1)
    g1 = _gather_rows(table1, nidx1.reshape(-1))
    cpad1 = jnp.concatenate(
        [jnp.zeros((B, SP1, 3), F32), s1x[..., None], s1y[..., None],
         s1z[..., None], jnp.zeros((B, SP1, 10), F32)], axis=-1)
    (w11, b11), (w12, b12), (w13, b13) = sa1
    x1 = _sa_mlp_max(g1.reshape(B, SP1 * K, 16), cpad1, vc1,
                     _pad_rows(w11, 16), b11, w12, b12, w13, b13,
                     SP=SP1, SBLK=128)

    idx2, s2x, s2y, s2z = _fps(s1x, s1y, s1z, S1, S2, SP2)
    nidx2, vc2 = _ball_topk(s2x, s2y, s2z, s1x, s1y, s1z,
                            r2=np.float32(0.4 * 0.4), n_real=S1, stride=SP1)
    table2 = jnp.concatenate(
        [x1.reshape(B * SP1, 128), s1x.reshape(B * SP1, 1),
         s1y.reshape(B * SP1, 1), s1z.reshape(B * SP1, 1),
         jnp.zeros((B * SP1, 13), F32)], axis=1)
    g2 = _gather_rows(table2, nidx2.reshape(-1))
    cpad2 = jnp.concatenate(
        [jnp.zeros((B, SP2, 128), F32), s2x[..., None], s2y[..., None],
         s2z[..., None], jnp.zeros((B, SP2, 13), F32)], axis=-1)
    (w21, b21), (w22, b22), (w23, b23) = sa2
    x2 = _sa_mlp_max(g2.reshape(B, SP2 * K, 144), cpad2, vc2,
                     _pad_rows(w21, 144), b21, w22, b22, w23, b23,
                     SP=SP2, SBLK=128)

    outb = _tail(x2, s2x, s2y, s2z, s1x, s1y, s1z, pbx, pby, pbz, x1, xb,
                 sa3, fp3, fp2, fp1, head)

    out = outb.reshape(N, 32)
    gidx = (idx1[:, :S1]
            + jnp.arange(B, dtype=jnp.int32)[:, None] * P).reshape(-1)
    return out, gidx
